# baseline (speedup 1.0000x reference)
import functools

import jax
import jax.numpy as jnp
from jax import lax
from jax.experimental import pallas as pl
from jax.experimental.pallas import tpu as pltpu
from jax.experimental.pallas import tpu_sc as plsc

NC = 2
NS = 16
NW = NC * NS
NBUF = 8


@jax.jit
def _embed(x, weight):
    batch, fields = x.shape
    dim = weight.shape[1]
    r_per_w = batch // NW
    n_groups = r_per_w // NBUF

    mesh = plsc.VectorSubcoreMesh(core_axis_name="c", subcore_axis_name="s")

    @functools.partial(
        pl.kernel,
        out_type=jax.ShapeDtypeStruct((batch, fields, dim), jnp.float32),
        mesh=mesh,
        scratch_types=[
            pltpu.VMEM((r_per_w, fields), jnp.int32),
            [pltpu.VMEM((fields, dim), jnp.float32) for _ in range(NBUF)],
            [pltpu.SemaphoreType.DMA for _ in range(NBUF)],
            [pltpu.SemaphoreType.DMA for _ in range(NBUF)],
        ],
        compiler_params=pltpu.CompilerParams(use_tc_tiling_on_sc=False),
    )
    def emb_kernel(idx_hbm, table_hbm, out_hbm, idx_v, bufs, gsems, ssems):
        wid = lax.axis_index("s") * NC + lax.axis_index("c")
        rbase = wid * r_per_w
        pltpu.sync_copy(idx_hbm.at[pl.ds(rbase, r_per_w)], idx_v)

        for b in range(NBUF):
            pltpu.async_copy(table_hbm.at[idx_v.at[b]], bufs[b], gsems[b])

        def group(g, carry):
            j0 = g * NBUF
            for b in range(NBUF):
                pltpu.make_async_copy(
                    table_hbm.at[idx_v.at[j0 + b]], bufs[b], gsems[b]
                ).wait()
                pltpu.async_copy(bufs[b], out_hbm.at[rbase + j0 + b], ssems[b])
            for b in range(NBUF):
                pltpu.make_async_copy(
                    bufs[b], out_hbm.at[rbase + j0 + b], ssems[b]
                ).wait()

                @pl.when(g + 1 < n_groups)
                def _():
                    pltpu.async_copy(
                        table_hbm.at[idx_v.at[j0 + NBUF + b]], bufs[b], gsems[b]
                    )

            return carry

        lax.fori_loop(0, n_groups, group, 0)

    return emb_kernel(x, weight)


def kernel(x, weight):
    return _embed(x, weight)


# speedup vs baseline: 1.0316x; 1.0316x over previous
import functools

import jax
import jax.numpy as jnp
from jax import lax
from jax.experimental import pallas as pl
from jax.experimental.pallas import tpu as pltpu
from jax.experimental.pallas import tpu_sc as plsc

NC = 2
NS = 16
NW = NC * NS
GRP = 8
NBUF = 4
LANES = 16


@jax.jit
def _embed(xt, weight):
    fields, batch = xt.shape
    dim = weight.shape[1]
    r_per_w = batch // NW
    n_groups = r_per_w // GRP

    mesh = plsc.VectorSubcoreMesh(core_axis_name="c", subcore_axis_name="s")

    @functools.partial(
        pl.kernel,
        out_type=jax.ShapeDtypeStruct((batch, fields, dim), jnp.float32),
        mesh=mesh,
        scratch_types=[
            pltpu.VMEM((fields, r_per_w), jnp.int32),
            pltpu.VMEM((r_per_w, fields), jnp.int32),
            [pltpu.VMEM((GRP, fields, dim), jnp.float32) for _ in range(NBUF)],
            [pltpu.SemaphoreType.DMA for _ in range(NBUF)],
            [pltpu.SemaphoreType.DMA for _ in range(NBUF)],
        ],
        compiler_params=pltpu.CompilerParams(
            use_tc_tiling_on_sc=False, needs_layout_passes=False
        ),
    )
    def emb_kernel(xt_hbm, table_hbm, out_hbm, xt_v, idx_v, bufs, gsems, ssems):
        wid = lax.axis_index("s") * NC + lax.axis_index("c")
        rbase = wid * r_per_w
        pltpu.sync_copy(xt_hbm.at[:, pl.ds(rbase, r_per_w)], xt_v)

        lane = lax.iota(jnp.int32, LANES)

        def tr_field(f, carry):
            col = jnp.full((LANES,), f, jnp.int32)
            for k in range(r_per_w // LANES):
                vals = xt_v[f, pl.ds(k * LANES, LANES)]
                plsc.store_scatter(idx_v, [lane + k * LANES, col], vals)
            return carry

        lax.fori_loop(0, fields, tr_field, 0)

        def gathers(g, b):
            for r in range(GRP):
                pltpu.async_copy(
                    table_hbm.at[idx_v.at[g * GRP + r]], bufs[b].at[r], gsems[b]
                )

        def drain_gathers(g, b):
            for r in range(GRP):
                pltpu.make_async_copy(
                    table_hbm.at[idx_v.at[g * GRP + r]], bufs[b].at[r], gsems[b]
                ).wait()

        for b in range(NBUF):
            gathers(b, b)

        def group(g, carry):
            for b in range(NBUF):
                drain_gathers(g * NBUF + b, b)
                pltpu.async_copy(
                    bufs[b],
                    out_hbm.at[pl.ds(rbase + (g * NBUF + b) * GRP, GRP)],
                    ssems[b],
                )
            for b in range(NBUF):
                pltpu.make_async_copy(
                    bufs[b],
                    out_hbm.at[pl.ds(rbase + (g * NBUF + b) * GRP, GRP)],
                    ssems[b],
                ).wait()

                @pl.when(g + 1 < n_groups // NBUF)
                def _():
                    gathers((g + 1) * NBUF + b, b)

            return carry

        lax.fori_loop(0, n_groups // NBUF, group, 0)

    return emb_kernel(xt, weight)


def kernel(x, weight):
    return _embed(jnp.swapaxes(x, 0, 1), weight)
